# baseline (speedup 1.0000x reference)
import jax, jax.numpy as jnp
from jax.experimental import pallas as pl


def kernel(x, edge_index, W_down0, b_down0, W_down1, b_down1, W_down2, b_down2,
           W_down3, b_down3, w_pool0, w_pool1, w_pool2,
           W_up0, b_up0, W_up1, b_up1, W_up2, b_up2):
    n0 = x.shape[0]
    k1 = 5000
    src, dst = edge_index[0], edge_index[1]
    s0 = jnp.tanh((x @ w_pool0) / jnp.sqrt(jnp.sum(w_pool0 * w_pool0)))
    sv0, perm0 = jax.lax.top_k(s0, k1)
    slot = jnp.full((n0,), k1, jnp.int32).at[perm0].set(jnp.arange(k1, dtype=jnp.int32))
    r_e = jnp.concatenate([slot[src], jnp.arange(k1, dtype=jnp.int32)])
    b_cols = jnp.concatenate([dst, perm0])
    B = jnp.zeros((k1 + 1, n0), jnp.float32).at[r_e, b_cols].add(1.0)[:k1]
    c_e = jnp.concatenate([slot[dst], jnp.arange(k1, dtype=jnp.int32)])
    c_rows = jnp.concatenate([src, perm0])
    C = jnp.zeros((n0, k1 + 1), jnp.float32).at[c_rows, c_e].add(1.0)[:, :k1]
    return jnp.sum(B, axis=0)[:5000] + jnp.sum(C, axis=0)


# speedup vs baseline: 3.7508x; 3.7508x over previous
import jax, jax.numpy as jnp
from jax.experimental import pallas as pl


def kernel(x, edge_index, W_down0, b_down0, W_down1, b_down1, W_down2, b_down2,
           W_down3, b_down3, w_pool0, w_pool1, w_pool2,
           W_up0, b_up0, W_up1, b_up1, W_up2, b_up2):
    n0 = x.shape[0]
    k1 = 5000
    A = jnp.zeros((n0, n0), jnp.float32).at[edge_index[0], edge_index[1]].add(1.0)
    A1 = (A + jnp.eye(n0, dtype=jnp.float32)).astype(jnp.bfloat16)
    s0 = jnp.tanh((x @ w_pool0) / jnp.sqrt(jnp.sum(w_pool0 * w_pool0)))
    sv0, perm0 = jax.lax.top_k(s0, k1)
    Brow = A1[perm0]
    Ccol = A1[:, perm0]
    return jnp.sum(Brow, axis=0) + jnp.sum(Ccol, axis=1)
